# baseline (speedup 1.0000x reference)
import jax
import jax.numpy as jnp
from jax import lax
from jax.experimental import pallas as pl
from jax.experimental.pallas import tpu as pltpu
from jax.experimental.pallas import tpu_sc as plsc

N = 10000
NI = 320000
D = 128
G = 64
CW = 16

NC = 2
NS = 16
NW = NC * NS
CHUNK = 128
N_CHUNKS = 80
PER_TILE = CHUNK * N_CHUNKS
NI_PAD = PER_TILE * NW
NP = 10240
PAD_ROW = N
ROWS_PER_TILE = NP // NS

_MESH = plsc.VectorSubcoreMesh(core_axis_name="c", subcore_axis_name="s")

_f32 = jnp.float32


def _phase_body(table, src, dst, zeros_nd, zeros_cw,
                out, cnt_out,
                sv0, sv1, sv2, sv3, dv0, dv1, dv2, dv3, rv0, rv1, ones_v,
                acc_sh, cd_sh,
                si0, si1, si2, si3, sg0, sg1, ss0, ss1):
    src_v = (sv0, sv1, sv2, sv3)
    dst_v = (dv0, dv1, dv2, dv3)
    rows_v = (rv0, rv1)
    sem_i = (si0, si1, si2, si3)
    sem_g = (sg0, sg1)
    sem_s = (ss0, ss1)

    cid = lax.axis_index("c")
    sid = lax.axis_index("s")
    wid = cid * NS + sid

    @pl.when(sid == 0)
    def _():
        pltpu.sync_copy(zeros_nd, acc_sh)
        pltpu.sync_copy(zeros_cw, cd_sh)

    for r in range(CHUNK):
        ones_v[r, :] = jnp.ones((CW,), _f32)

    plsc.subcore_barrier()

    def issue_idx(j, c):
        base = wid * PER_TILE + c * CHUNK
        pltpu.async_copy(src.at[pl.ds(base, CHUNK)], src_v[j], sem_i[j])
        pltpu.async_copy(dst.at[pl.ds(base, CHUNK)], dst_v[j], sem_i[j])

    def wait_idx(j):
        pltpu.make_async_copy(src.at[pl.ds(0, CHUNK)], src_v[j], sem_i[j]).wait()
        pltpu.make_async_copy(dst.at[pl.ds(0, CHUNK)], dst_v[j], sem_i[j]).wait()

    def issue_scatter(j, b):
        pltpu.async_copy(rows_v[b], acc_sh.at[dst_v[j]], sem_s[b], add=True)
        pltpu.async_copy(ones_v, cd_sh.at[dst_v[j]], sem_s[b], add=True)

    def wait_scatter(j, b):
        pltpu.make_async_copy(rows_v[b], acc_sh.at[dst_v[j]], sem_s[b]).wait()
        pltpu.make_async_copy(ones_v, cd_sh.at[dst_v[j]], sem_s[b]).wait()

    def wait_gather(j, b):
        pltpu.make_async_copy(table.at[src_v[j]], rows_v[b], sem_g[b]).wait()

    issue_idx(0, 0)
    issue_idx(1, 1)
    n_super = N_CHUNKS // 4

    def body(s, carry):
        for j in range(4):
            c = 4 * s + j
            b = j % 2
            wait_idx(j)

            @pl.when(c >= 2)
            def _(j=j, b=b):
                wait_scatter((j + 2) % 4, b)

            @pl.when(c + 2 < N_CHUNKS)
            def _(j=j, c=c):
                issue_idx((j + 2) % 4, c + 2)

            pltpu.async_copy(table.at[src_v[j]], rows_v[b], sem_g[b])

            @pl.when(c >= 1)
            def _(j=j, b=b):
                wait_gather((j + 3) % 4, 1 - b)
                issue_scatter((j + 3) % 4, 1 - b)

        return carry

    lax.fori_loop(0, n_super, body, 0)

    j_last = (N_CHUNKS - 1) % 4
    b_last = (N_CHUNKS - 1) % 2
    wait_gather(j_last, b_last)
    issue_scatter(j_last, b_last)
    wait_scatter((N_CHUNKS - 2) % 4, (N_CHUNKS - 2) % 2)
    wait_scatter(j_last, b_last)

    plsc.subcore_barrier()

    r0 = sid * ROWS_PER_TILE

    def wb(k, carry):
        pltpu.sync_copy(acc_sh.at[pl.ds(r0 + k * CHUNK, CHUNK)], rows_v[0])
        pltpu.sync_copy(rows_v[0],
                        out.at[pl.ds(cid * NP + r0 + k * CHUNK, CHUNK)])
        pltpu.sync_copy(cd_sh.at[pl.ds(r0 + k * CHUNK, CHUNK)], ones_v)
        pltpu.sync_copy(ones_v,
                        cnt_out.at[pl.ds(cid * NP + r0 + k * CHUNK, CHUNK)])
        return carry

    lax.fori_loop(0, ROWS_PER_TILE // CHUNK, wb, 0)


_phase = pl.kernel(
    _phase_body,
    out_type=(
        jax.ShapeDtypeStruct((NC * NP, D), _f32),
        jax.ShapeDtypeStruct((NC * NP, CW), _f32),
    ),
    mesh=_MESH,
    scratch_types=(
        [pltpu.VMEM((CHUNK,), jnp.int32)] * 8
        + [pltpu.VMEM((CHUNK, D), _f32)] * 2
        + [pltpu.VMEM((CHUNK, CW), _f32)]
        + [pltpu.VMEM_SHARED((NP, D), _f32),
           pltpu.VMEM_SHARED((NP, CW), _f32)]
        + [pltpu.SemaphoreType.DMA] * 8
    ),
    compiler_params=pltpu.CompilerParams(use_tc_tiling_on_sc=False),
)


_RB = 1000
_NB = N // _RB


def _tc_matmul(x, W):
    def body(x_ref, w_ref, o_ref):
        o_ref[...] = jnp.dot(x_ref[...], w_ref[...],
                             preferred_element_type=_f32)

    return pl.pallas_call(
        body,
        grid=(_NB,),
        in_specs=[pl.BlockSpec((_RB, D), lambda i: (i, 0)),
                  pl.BlockSpec((D, D), lambda i: (0, 0))],
        out_specs=pl.BlockSpec((_RB, D), lambda i: (i, 0)),
        out_shape=jax.ShapeDtypeStruct((N, D), _f32),
    )(x, W)


def _tc_combine(partials, cnts, bias=None, relu=False):
    p3 = partials.reshape(NC, NP, D)
    c3 = cnts.reshape(NC, NP, CW)

    def body(*refs):
        if bias is None:
            p_ref, c_ref, o_ref = refs
        else:
            p_ref, c_ref, b_ref, o_ref = refs
        s = p_ref[0] + p_ref[1]
        cnt = c_ref[0, :, 0:1] + c_ref[1, :, 0:1]
        inv = jnp.where(cnt > 0.0, 1.0 / cnt, 0.0)
        r = s * inv
        if bias is not None:
            r = r + b_ref[...]
        if relu:
            r = jnp.maximum(r, 0.0)
        o_ref[...] = r

    in_specs = [pl.BlockSpec((NC, _RB, D), lambda i: (0, i, 0)),
                pl.BlockSpec((NC, _RB, CW), lambda i: (0, i, 0))]
    args = [p3, c3]
    if bias is not None:
        in_specs.append(pl.BlockSpec((1, D), lambda i: (0, 0)))
        args.append(bias.reshape(1, D))

    return pl.pallas_call(
        body,
        grid=(_NB,),
        in_specs=in_specs,
        out_specs=pl.BlockSpec((_RB, D), lambda i: (i, 0)),
        out_shape=jax.ShapeDtypeStruct((N, D), _f32),
    )(*args)


def _tc_pool(h, batch3d):
    def body(h_ref, b_ref, o_ref, sums, cnts):
        i = pl.program_id(0)

        @pl.when(i == 0)
        def _():
            sums[...] = jnp.zeros_like(sums)
            cnts[...] = jnp.zeros_like(cnts)

        b = b_ref[0, 0, :]
        mask = (b[:, None] == lax.broadcasted_iota(jnp.int32, (_RB, G), 1)
                ).astype(_f32)
        sums[...] += lax.dot_general(mask, h_ref[...],
                                     (((0,), (0,)), ((), ())),
                                     preferred_element_type=_f32)
        cnts[...] += jnp.broadcast_to(jnp.sum(mask, axis=0)[:, None], (G, D))

        @pl.when(i == _NB - 1)
        def _():
            o_ref[...] = sums[...] / jnp.maximum(cnts[...], 1.0)

    return pl.pallas_call(
        body,
        grid=(_NB,),
        in_specs=[pl.BlockSpec((_RB, D), lambda i: (i, 0)),
                  pl.BlockSpec((1, 1, _RB), lambda i: (i, 0, 0))],
        out_specs=pl.BlockSpec((G, D), lambda i: (0, 0)),
        out_shape=jax.ShapeDtypeStruct((G, D), _f32),
        scratch_shapes=[pltpu.VMEM((G, D), _f32), pltpu.VMEM((G, D), _f32)],
    )(h, batch3d)


def kernel(x, hyperedge_index, batch, W1, b1, W2, b2):
    node_idx = hyperedge_index[0].astype(jnp.int32)
    edge_idx = hyperedge_index[1].astype(jnp.int32)
    batch3d = batch.astype(jnp.int32).reshape(_NB, 1, _RB)

    n_pad = NI_PAD - NI
    src_pad = jnp.zeros((n_pad,), jnp.int32)
    dst_pad = jnp.full((n_pad,), PAD_ROW, jnp.int32)
    node_src = jnp.concatenate([node_idx, src_pad])
    node_dst = jnp.concatenate([node_idx, dst_pad])
    edge_src = jnp.concatenate([edge_idx, src_pad])
    edge_dst = jnp.concatenate([edge_idx, dst_pad])

    zeros_nd = jnp.zeros((NP, D), _f32)
    zeros_cw = jnp.zeros((NP, CW), _f32)

    xw = _tc_matmul(x, W1)
    heP, cntB = _phase(xw, node_src, edge_dst, zeros_nd, zeros_cw)
    he = _tc_combine(heP, cntB)
    outP, cntD = _phase(he, edge_src, node_dst, zeros_nd, zeros_cw)
    h = _tc_combine(outP, cntD, bias=b1, relu=True)

    xw = _tc_matmul(h, W2)
    heP, cntB = _phase(xw, node_src, edge_dst, zeros_nd, zeros_cw)
    he = _tc_combine(heP, cntB)
    outP, cntD = _phase(he, edge_src, node_dst, zeros_nd, zeros_cw)
    h = _tc_combine(outP, cntD, bias=b2, relu=True)

    return _tc_pool(h, batch3d)


# speedup vs baseline: 3.5480x; 3.5480x over previous
import jax
import jax.numpy as jnp
from jax import lax
from jax.experimental import pallas as pl
from jax.experimental.pallas import tpu as pltpu
from jax.experimental.pallas import tpu_sc as plsc

N = 10000
NI = 320000
D = 128
G = 64
CW = 16

NC = 2
NS = 16
NW = NC * NS
CHUNK = 128
N_CHUNKS = 80
PER_TILE = CHUNK * N_CHUNKS
NI_PAD = PER_TILE * NW
NP = 10240
PAD_ROW = N
ROWS_PER_TILE = NP // NS

_MESH = plsc.VectorSubcoreMesh(core_axis_name="c", subcore_axis_name="s")

_f32 = jnp.float32


def _phase_body(table, src, dst, zeros_nd, zeros_cw,
                out, cnt_out,
                sv0, sv1, sv2, sv3, dv0, dv1, dv2, dv3, rv0, rv1, ones_v,
                acc_sh, cd_sh,
                si0, si1, si2, si3, sg0, sg1, ss0, ss1):
    src_v = (sv0, sv1, sv2, sv3)
    dst_v = (dv0, dv1, dv2, dv3)
    rows_v = (rv0, rv1)
    sem_i = (si0, si1, si2, si3)
    sem_g = (sg0, sg1)
    sem_s = (ss0, ss1)

    cid = lax.axis_index("c")
    sid = lax.axis_index("s")
    wid = cid * NS + sid

    @pl.when(sid == 0)
    def _():
        pltpu.sync_copy(zeros_nd, acc_sh)
        pltpu.sync_copy(zeros_cw, cd_sh)

    for r in range(CHUNK):
        ones_v[r, :] = jnp.ones((CW,), _f32)

    plsc.subcore_barrier()

    def issue_idx(j, c):
        base = wid * PER_TILE + c * CHUNK
        pltpu.async_copy(src.at[pl.ds(base, CHUNK)], src_v[j], sem_i[j])
        pltpu.async_copy(dst.at[pl.ds(base, CHUNK)], dst_v[j], sem_i[j])

    def wait_idx(j):
        pltpu.make_async_copy(src.at[pl.ds(0, CHUNK)], src_v[j], sem_i[j]).wait()
        pltpu.make_async_copy(dst.at[pl.ds(0, CHUNK)], dst_v[j], sem_i[j]).wait()

    def issue_scatter(j, b):
        pltpu.async_copy(rows_v[b], acc_sh.at[dst_v[j]], sem_s[b], add=True)
        pltpu.async_copy(ones_v, cd_sh.at[dst_v[j]], sem_s[b], add=True)

    def wait_scatter(j, b):
        pltpu.make_async_copy(rows_v[b], acc_sh.at[dst_v[j]], sem_s[b]).wait()
        pltpu.make_async_copy(ones_v, cd_sh.at[dst_v[j]], sem_s[b]).wait()

    def wait_gather(j, b):
        pltpu.make_async_copy(table.at[src_v[j]], rows_v[b], sem_g[b]).wait()

    issue_idx(0, 0)
    issue_idx(1, 1)
    n_super = N_CHUNKS // 4

    def body(s, carry):
        for j in range(4):
            c = 4 * s + j
            b = j % 2
            wait_idx(j)

            @pl.when(c >= 2)
            def _(j=j, b=b):
                wait_scatter((j + 2) % 4, b)

            @pl.when(c + 2 < N_CHUNKS)
            def _(j=j, c=c):
                issue_idx((j + 2) % 4, c + 2)

            pltpu.async_copy(table.at[src_v[j]], rows_v[b], sem_g[b])

            @pl.when(c >= 1)
            def _(j=j, b=b):
                wait_gather((j + 3) % 4, 1 - b)
                issue_scatter((j + 3) % 4, 1 - b)

        return carry

    lax.fori_loop(0, n_super, body, 0)

    j_last = (N_CHUNKS - 1) % 4
    b_last = (N_CHUNKS - 1) % 2
    wait_gather(j_last, b_last)
    issue_scatter(j_last, b_last)
    wait_scatter((N_CHUNKS - 2) % 4, (N_CHUNKS - 2) % 2)
    wait_scatter(j_last, b_last)

    plsc.subcore_barrier()

    r0 = sid * ROWS_PER_TILE

    def wb(k, carry):
        pltpu.sync_copy(acc_sh.at[pl.ds(r0 + k * CHUNK, CHUNK)], rows_v[0])
        pltpu.sync_copy(rows_v[0],
                        out.at[pl.ds(cid * NP + r0 + k * CHUNK, CHUNK)])
        pltpu.sync_copy(cd_sh.at[pl.ds(r0 + k * CHUNK, CHUNK)], ones_v)
        pltpu.sync_copy(ones_v,
                        cnt_out.at[pl.ds(cid * NP + r0 + k * CHUNK, CHUNK)])
        return carry

    lax.fori_loop(0, ROWS_PER_TILE // CHUNK, wb, 0)


_phase = pl.kernel(
    _phase_body,
    out_type=(
        jax.ShapeDtypeStruct((NC * NP, D), _f32),
        jax.ShapeDtypeStruct((NC * NP, CW), _f32),
    ),
    mesh=_MESH,
    scratch_types=(
        [pltpu.VMEM((CHUNK,), jnp.int32)] * 8
        + [pltpu.VMEM((CHUNK, D), _f32)] * 2
        + [pltpu.VMEM((CHUNK, CW), _f32)]
        + [pltpu.VMEM_SHARED((NP, D), _f32),
           pltpu.VMEM_SHARED((NP, CW), _f32)]
        + [pltpu.SemaphoreType.DMA] * 8
    ),
    compiler_params=pltpu.CompilerParams(use_tc_tiling_on_sc=False),
)


_RB = 1000
_NB = N // _RB


def _tc_matmul(x, W):
    def body(x_ref, w_ref, o_ref):
        o_ref[...] = jnp.dot(x_ref[...], w_ref[...],
                             preferred_element_type=_f32)

    return pl.pallas_call(
        body,
        grid=(_NB,),
        in_specs=[pl.BlockSpec((_RB, D), lambda i: (i, 0)),
                  pl.BlockSpec((D, D), lambda i: (0, 0))],
        out_specs=pl.BlockSpec((_RB, D), lambda i: (i, 0)),
        out_shape=jax.ShapeDtypeStruct((N, D), _f32),
    )(x, W)


def _tc_combine(partials, cnts, bias=None, relu=False):
    p3 = partials.reshape(NC, NP, D)
    c3 = cnts.reshape(NC, NP, CW)

    def body(*refs):
        if bias is None:
            p_ref, c_ref, o_ref = refs
        else:
            p_ref, c_ref, b_ref, o_ref = refs
        s = p_ref[0] + p_ref[1]
        cnt = c_ref[0, :, 0:1] + c_ref[1, :, 0:1]
        inv = jnp.where(cnt > 0.0, 1.0 / cnt, 0.0)
        r = s * inv
        if bias is not None:
            r = r + b_ref[...]
        if relu:
            r = jnp.maximum(r, 0.0)
        o_ref[...] = r

    in_specs = [pl.BlockSpec((NC, _RB, D), lambda i: (0, i, 0)),
                pl.BlockSpec((NC, _RB, CW), lambda i: (0, i, 0))]
    args = [p3, c3]
    if bias is not None:
        in_specs.append(pl.BlockSpec((1, D), lambda i: (0, 0)))
        args.append(bias.reshape(1, D))

    return pl.pallas_call(
        body,
        grid=(_NB,),
        in_specs=in_specs,
        out_specs=pl.BlockSpec((_RB, D), lambda i: (i, 0)),
        out_shape=jax.ShapeDtypeStruct((N, D), _f32),
    )(*args)


def _tc_pool(h, batch3d):
    def body(h_ref, b_ref, o_ref, sums, cnts):
        i = pl.program_id(0)

        @pl.when(i == 0)
        def _():
            sums[...] = jnp.zeros_like(sums)
            cnts[...] = jnp.zeros_like(cnts)

        b = b_ref[0, 0, :]
        mask = (b[:, None] == lax.broadcasted_iota(jnp.int32, (_RB, G), 1)
                ).astype(_f32)
        sums[...] += lax.dot_general(mask, h_ref[...],
                                     (((0,), (0,)), ((), ())),
                                     preferred_element_type=_f32)
        cnts[...] += jnp.broadcast_to(jnp.sum(mask, axis=0)[:, None], (G, D))

        @pl.when(i == _NB - 1)
        def _():
            o_ref[...] = sums[...] / jnp.maximum(cnts[...], 1.0)

    return pl.pallas_call(
        body,
        grid=(_NB,),
        in_specs=[pl.BlockSpec((_RB, D), lambda i: (i, 0)),
                  pl.BlockSpec((1, 1, _RB), lambda i: (i, 0, 0))],
        out_specs=pl.BlockSpec((G, D), lambda i: (0, 0)),
        out_shape=jax.ShapeDtypeStruct((G, D), _f32),
        scratch_shapes=[pltpu.VMEM((G, D), _f32), pltpu.VMEM((G, D), _f32)],
    )(h, batch3d)


def kernel(x, hyperedge_index, batch, W1, b1, W2, b2):
    node_idx = hyperedge_index[0].astype(jnp.int32)
    edge_idx = hyperedge_index[1].astype(jnp.int32)
    batch3d = batch.astype(jnp.int32).reshape(_NB, 1, _RB)

    n_pad = NI_PAD - NI
    src_pad = jnp.arange(n_pad, dtype=jnp.int32) % N
    dst_pad = PAD_ROW + (jnp.arange(n_pad, dtype=jnp.int32) % (NP - N))
    node_src = jnp.concatenate([node_idx, src_pad])
    node_dst = jnp.concatenate([node_idx, dst_pad])
    edge_src = jnp.concatenate([edge_idx, src_pad])
    edge_dst = jnp.concatenate([edge_idx, dst_pad])

    zeros_nd = jnp.zeros((NP, D), _f32)
    zeros_cw = jnp.zeros((NP, CW), _f32)

    xw = _tc_matmul(x, W1)
    heP, cntB = _phase(xw, node_src, edge_dst, zeros_nd, zeros_cw)
    he = _tc_combine(heP, cntB)
    outP, cntD = _phase(he, edge_src, node_dst, zeros_nd, zeros_cw)
    h = _tc_combine(outP, cntD, bias=b1, relu=True)

    xw = _tc_matmul(h, W2)
    heP, cntB = _phase(xw, node_src, edge_dst, zeros_nd, zeros_cw)
    he = _tc_combine(heP, cntB)
    outP, cntD = _phase(he, edge_src, node_dst, zeros_nd, zeros_cw)
    h = _tc_combine(outP, cntD, bias=b2, relu=True)

    return _tc_pool(h, batch3d)
